# baseline (speedup 1.0000x reference)
import functools

import jax
import jax.numpy as jnp
from jax import lax
from jax.experimental import pallas as pl
from jax.experimental.pallas import tpu as pltpu
from jax.experimental.pallas import tpu_sc as plsc

VOCAB = 1000000
DIM = 64
B_TOTAL = 4096 * 200
NUM_WORKERS = 32
B_PER_W = B_TOTAL // NUM_WORKERS
CHUNK = 128
NCHUNKS = B_PER_W // CHUNK
NBUF = 8
NROUNDS = NCHUNKS // NBUF

_mesh = plsc.VectorSubcoreMesh(core_axis_name="c", subcore_axis_name="s")


@functools.partial(
    pl.kernel,
    out_type=jax.ShapeDtypeStruct((B_TOTAL, DIM), jnp.float32),
    mesh=_mesh,
    scratch_types=[
        pltpu.VMEM((NBUF, CHUNK), jnp.int32),
        pltpu.VMEM((NBUF, CHUNK, DIM), jnp.float32),
        pltpu.SemaphoreType.DMA((NBUF,)),
        pltpu.SemaphoreType.DMA((NBUF,)),
    ],
    compiler_params=pltpu.CompilerParams(use_tc_tiling_on_sc=False),
)
def _embed_gather(x_hbm, table_hbm, out_hbm, idx_v, rows_v, gsem, wsem):
    wid = lax.axis_index("s") * 2 + lax.axis_index("c")
    base = pl.multiple_of(wid * B_PER_W, CHUNK)

    def chunk_off(c):
        return pl.multiple_of(base + c * CHUNK, CHUNK)

    def load_idx(b, c):
        pltpu.sync_copy(x_hbm.at[pl.ds(chunk_off(c), CHUNK)], idx_v.at[b])

    def fire_gather(b):
        pltpu.async_copy(table_hbm.at[idx_v.at[b]], rows_v.at[b], gsem.at[b])

    def wait_gather(b):
        pltpu.make_async_copy(
            table_hbm.at[idx_v.at[b]], rows_v.at[b], gsem.at[b]
        ).wait()

    def fire_writeback(b, c):
        pltpu.async_copy(
            rows_v.at[b], out_hbm.at[pl.ds(chunk_off(c), CHUNK)], wsem.at[b]
        )

    def wait_writeback(b, c):
        pltpu.make_async_copy(
            rows_v.at[b], out_hbm.at[pl.ds(chunk_off(c), CHUNK)], wsem.at[b]
        ).wait()

    for b in range(NBUF):
        load_idx(b, b)
        fire_gather(b)

    def round_body(r, carry):
        c0 = r * NBUF
        for b in range(NBUF):
            wait_gather(b)
            fire_writeback(b, c0 + b)
        for b in range(NBUF):
            load_idx(b, c0 + b + NBUF)
            wait_writeback(b, c0 + b)
            fire_gather(b)
        return carry

    lax.fori_loop(0, NROUNDS - 1, round_body, 0, unroll=False)

    c0 = (NROUNDS - 1) * NBUF
    for b in range(NBUF):
        wait_gather(b)
        fire_writeback(b, c0 + b)
    for b in range(NBUF):
        wait_writeback(b, c0 + b)


def kernel(x, table):
    flat_x = x.reshape(-1).astype(jnp.int32)
    out = _embed_gather(flat_x, table)
    return out.reshape(x.shape[0], x.shape[1], DIM)


# speedup vs baseline: 1.0259x; 1.0259x over previous
import functools

import jax
import jax.numpy as jnp
from jax import lax
from jax.experimental import pallas as pl
from jax.experimental.pallas import tpu as pltpu
from jax.experimental.pallas import tpu_sc as plsc

VOCAB = 1000000
DIM = 64
B_TOTAL = 4096 * 200
NUM_WORKERS = 32
B_PER_W = B_TOTAL // NUM_WORKERS
CHUNK = 128
NCHUNKS = B_PER_W // CHUNK
NBUF = 8
NROUNDS = NCHUNKS // NBUF

_mesh = plsc.VectorSubcoreMesh(core_axis_name="c", subcore_axis_name="s")


@functools.partial(
    pl.kernel,
    out_type=jax.ShapeDtypeStruct((B_TOTAL, DIM), jnp.float32),
    mesh=_mesh,
    scratch_types=[
        pltpu.VMEM((B_PER_W,), jnp.int32),
        pltpu.VMEM((NBUF, CHUNK, DIM), jnp.float32),
        pltpu.SemaphoreType.DMA,
        pltpu.SemaphoreType.DMA((NBUF,)),
        pltpu.SemaphoreType.DMA((NBUF,)),
    ],
    compiler_params=pltpu.CompilerParams(use_tc_tiling_on_sc=False),
)
def _embed_gather(x_hbm, table_hbm, out_hbm, idx_v, rows_v, isem, gsem, wsem):
    wid = lax.axis_index("s") * 2 + lax.axis_index("c")
    base = pl.multiple_of(wid * B_PER_W, CHUNK)

    def chunk_off(c):
        return pl.multiple_of(base + c * CHUNK, CHUNK)

    def idx_slice(c):
        return idx_v.at[pl.ds(pl.multiple_of(c * CHUNK, CHUNK), CHUNK)]

    def fire_gather(b, c):
        pltpu.async_copy(table_hbm.at[idx_slice(c)], rows_v.at[b], gsem.at[b])

    def wait_gather(b, c):
        pltpu.make_async_copy(
            table_hbm.at[idx_slice(c)], rows_v.at[b], gsem.at[b]
        ).wait()

    def fire_writeback(b, c):
        pltpu.async_copy(
            rows_v.at[b], out_hbm.at[pl.ds(chunk_off(c), CHUNK)], wsem.at[b]
        )

    def wait_writeback(b, c):
        pltpu.make_async_copy(
            rows_v.at[b], out_hbm.at[pl.ds(chunk_off(c), CHUNK)], wsem.at[b]
        ).wait()

    pltpu.async_copy(x_hbm.at[pl.ds(base, B_PER_W)], idx_v, isem).wait()

    for b in range(NBUF):
        fire_gather(b, b)

    def round_body(r, carry):
        c0 = r * NBUF
        for b in range(NBUF):
            wait_gather(b, c0 + b)
            fire_writeback(b, c0 + b)
        for b in range(NBUF):
            wait_writeback(b, c0 + b)
            fire_gather(b, c0 + b + NBUF)
        return carry

    lax.fori_loop(0, NROUNDS - 1, round_body, 0, unroll=False)

    c0 = (NROUNDS - 1) * NBUF
    for b in range(NBUF):
        wait_gather(b, c0 + b)
        fire_writeback(b, c0 + b)
    for b in range(NBUF):
        wait_writeback(b, c0 + b)


def kernel(x, table):
    flat_x = x.reshape(-1).astype(jnp.int32)
    out = _embed_gather(flat_x, table)
    return out.reshape(x.shape[0], x.shape[1], DIM)
